# baseline (speedup 1.0000x reference)
import jax
import jax.numpy as jnp
from jax.experimental import pallas as pl
from jax.experimental.pallas import tpu as pltpu

_OFFSET = 2
_MAX_POS = 2048
_D_MODEL = 1024
_LANES = 256
_ROW_OFF = _OFFSET * _D_MODEL // _LANES
_ROWS_OUT = _MAX_POS * _D_MODEL // _LANES


def _dma_slice_kernel(in_hbm, out_hbm, sem):
    copy = pltpu.make_async_copy(
        in_hbm.at[pl.ds(_ROW_OFF, _ROWS_OUT), :], out_hbm, sem
    )
    copy.start()
    copy.wait()


def kernel(position_weights):
    flat = position_weights.reshape(_ROW_OFF + _ROWS_OUT, _LANES)
    out = pl.pallas_call(
        _dma_slice_kernel,
        in_specs=[pl.BlockSpec(memory_space=pl.ANY)],
        out_specs=pl.BlockSpec(memory_space=pl.ANY),
        scratch_shapes=[pltpu.SemaphoreType.DMA],
        out_shape=jax.ShapeDtypeStruct((_ROWS_OUT, _LANES), jnp.float32),
    )(flat)
    return out.reshape(_MAX_POS, _D_MODEL)


# speedup vs baseline: 30.0855x; 30.0855x over previous
import jax
import jax.numpy as jnp
from jax.experimental import pallas as pl
from jax.experimental.pallas import tpu as pltpu

_OFFSET = 2
_MAX_POS = 2048
_D_MODEL = 1024
_B = 256
_G = _MAX_POS // _B


def _shift_copy_kernel(big_ref, carry_ref, out_ref):
    out_ref[0 : _B - _OFFSET, :] = big_ref[_OFFSET:_B, :]
    out_ref[_B - _OFFSET : _B, :] = carry_ref[0:_OFFSET, :]


def kernel(position_weights):
    return pl.pallas_call(
        _shift_copy_kernel,
        grid=(_G,),
        in_specs=[
            pl.BlockSpec((_B, _D_MODEL), lambda i: (i, 0)),
            pl.BlockSpec((8, _D_MODEL), lambda i: ((i + 1) * (_B // 8), 0)),
        ],
        out_specs=pl.BlockSpec((_B, _D_MODEL), lambda i: (i, 0)),
        out_shape=jax.ShapeDtypeStruct((_MAX_POS, _D_MODEL), jnp.float32),
        compiler_params=pltpu.CompilerParams(
            dimension_semantics=("arbitrary",),
        ),
    )(position_weights, position_weights)
